# baseline (speedup 1.0000x reference)
import functools

import jax
import jax.numpy as jnp
from jax import lax
from jax.experimental import pallas as pl
from jax.experimental.pallas import tpu as pltpu
from jax.experimental.pallas import tpu_sc as plsc

_EPS = 1e-12
_Q = 2


def _sc_gather_slice(word_embeddings, flat_ids, q, b, s, hid, with_dummy):
    info = plsc.get_sparse_core_info()
    nw = info.num_cores * info.num_subcores
    sl = s // _Q
    n_tok = b * sl
    per_w = n_tok // nw
    chunk = per_w // 2
    wps = sl // per_w
    mesh = plsc.VectorSubcoreMesh(core_axis_name="c", subcore_axis_name="s")
    out_type = [jax.ShapeDtypeStruct((n_tok, hid), jnp.float32)]
    if with_dummy:
        out_type.append(jax.ShapeDtypeStruct((b, s, hid), jnp.float32))

    @functools.partial(
        pl.kernel,
        mesh=mesh,
        out_type=tuple(out_type) if with_dummy else out_type[0],
        scratch_types=[
            pltpu.VMEM((per_w,), jnp.int32),
            pltpu.VMEM((chunk, hid), jnp.float32),
            pltpu.VMEM((chunk, hid), jnp.float32),
            pltpu.SemaphoreType.DMA,
            pltpu.SemaphoreType.DMA,
        ],
    )
    def gather_k(table_hbm, idx_hbm, out_hbm, *rest):
        idx_v, buf0, buf1, gsem, wsem = rest[-5:]
        wid = lax.axis_index("s") * info.num_cores + lax.axis_index("c")
        bb = wid // wps
        pl0 = (wid % wps) * per_w
        src0 = bb * s + q * sl + pl0
        dst0 = bb * sl + pl0
        bufs = (buf0, buf1)
        pltpu.sync_copy(idx_hbm.at[pl.ds(src0, per_w)], idx_v)

        def start_gather(i):
            return pltpu.async_copy(
                table_hbm.at[idx_v.at[pl.ds(i * chunk, chunk)]], bufs[i % 2], gsem
            )

        gathers = [start_gather(0), start_gather(1)]
        writes = [None, None]
        n_chunks = per_w // chunk
        for i in range(n_chunks):
            gathers[i % 2].wait()
            writes[i % 2] = pltpu.async_copy(
                bufs[i % 2], out_hbm.at[pl.ds(dst0 + i * chunk, chunk)], wsem
            )
            if i + 2 < n_chunks:
                writes[i % 2].wait()
                gathers[i % 2] = start_gather(i + 2)
        writes[(n_chunks - 2) % 2].wait()
        writes[(n_chunks - 1) % 2].wait()

    return gather_k(word_embeddings, flat_ids)


def _ln_body(g_ref, pos_ref, tok_ref, w_ref, b_ref, obuf_ref, o_ref):
    x = g_ref[0] + pos_ref[...] + tok_ref[...]
    mean = jnp.mean(x, axis=-1, keepdims=True)
    xc = x - mean
    var = jnp.mean(xc * xc, axis=-1, keepdims=True)
    o_ref[0] = (xc * lax.rsqrt(var + _EPS)) * w_ref[...] + b_ref[...]


def kernel(input_ids, word_embeddings, position_embeddings, token_type_embeddings, ln_weight, ln_bias):
    b, s = input_ids.shape
    vocab, hid = word_embeddings.shape
    n = b * s
    sl = s // _Q
    flat_ids = input_ids.reshape(n).astype(jnp.int32)

    g0, out = _sc_gather_slice(word_embeddings, flat_ids, 0, b, s, hid, True)
    g1 = _sc_gather_slice(word_embeddings, flat_ids, 1, b, s, hid, False)
    gathered = [g0.reshape(b, sl, hid), g1.reshape(b, sl, hid)]

    tok = token_type_embeddings[0:1]
    w2 = ln_weight.reshape(1, hid)
    b2 = ln_bias.reshape(1, hid)

    bs = sl
    for q in range(_Q):
        pos_q = lax.slice_in_dim(position_embeddings, q * sl, (q + 1) * sl, axis=0)
        out = pl.pallas_call(
            _ln_body,
            grid=(b, sl // bs),
            in_specs=[
                pl.BlockSpec((1, bs, hid), lambda i, j: (i, j, 0)),
                pl.BlockSpec((bs, hid), lambda i, j: (j, 0)),
                pl.BlockSpec((1, hid), lambda i, j: (0, 0)),
                pl.BlockSpec((1, hid), lambda i, j: (0, 0)),
                pl.BlockSpec((1, hid), lambda i, j: (0, 0)),
                pl.BlockSpec(memory_space=pl.ANY),
            ],
            out_specs=pl.BlockSpec(
                (1, bs, hid), lambda i, j, q=q: (i, q * (sl // bs) + j, 0)
            ),
            out_shape=jax.ShapeDtypeStruct((b, s, hid), jnp.float32),
            input_output_aliases={5: 0},
        )(gathered[q], pos_q, tok, w2, b2, out)
    return out


# speedup vs baseline: 1.0674x; 1.0674x over previous
import functools

import jax
import jax.numpy as jnp
from jax import lax
from jax.experimental import pallas as pl
from jax.experimental.pallas import tpu as pltpu
from jax.experimental.pallas import tpu_sc as plsc

_EPS = 1e-12


def _sc_gather(word_embeddings, flat_ids, n_tokens, hid):
    info = plsc.get_sparse_core_info()
    nw = info.num_cores * info.num_subcores
    per_w = n_tokens // nw
    chunk = 64
    n_chunks = per_w // chunk
    mesh = plsc.VectorSubcoreMesh(core_axis_name="c", subcore_axis_name="s")

    @functools.partial(
        pl.kernel,
        mesh=mesh,
        out_type=jax.ShapeDtypeStruct((n_tokens, hid), jnp.float32),
        scratch_types=[
            pltpu.VMEM((per_w,), jnp.int32),
            pltpu.VMEM((chunk, hid), jnp.float32),
            pltpu.VMEM((chunk, hid), jnp.float32),
            pltpu.SemaphoreType.DMA,
            pltpu.SemaphoreType.DMA,
        ],
    )
    def gather_k(table_hbm, idx_hbm, out_hbm, idx_v, buf0, buf1, gsem, wsem):
        wid = lax.axis_index("s") * info.num_cores + lax.axis_index("c")
        base = wid * per_w
        bufs = (buf0, buf1)
        pltpu.sync_copy(idx_hbm.at[pl.ds(base, per_w)], idx_v)

        def start_gather(i):
            return pltpu.async_copy(
                table_hbm.at[idx_v.at[pl.ds(i * chunk, chunk)]], bufs[i % 2], gsem
            )

        gathers = [start_gather(0), start_gather(1)]
        writes = [None, None]
        for i in range(n_chunks):
            gathers[i % 2].wait()
            writes[i % 2] = pltpu.async_copy(
                bufs[i % 2], out_hbm.at[pl.ds(base + i * chunk, chunk)], wsem
            )
            if i + 2 < n_chunks:
                writes[i % 2].wait()
                gathers[i % 2] = start_gather(i + 2)
        writes[(n_chunks - 2) % 2].wait()
        writes[(n_chunks - 1) % 2].wait()

    return gather_k(word_embeddings, flat_ids)


def _ln_body(g_ref, pos_ref, tok_ref, w_ref, b_ref, o_ref):
    x = g_ref[0] + pos_ref[...] + tok_ref[...]
    mean = jnp.mean(x, axis=-1, keepdims=True)
    xc = x - mean
    var = jnp.mean(xc * xc, axis=-1, keepdims=True)
    o_ref[0] = (xc * lax.rsqrt(var + _EPS)) * w_ref[...] + b_ref[...]


def kernel(input_ids, word_embeddings, position_embeddings, token_type_embeddings, ln_weight, ln_bias):
    b, s = input_ids.shape
    vocab, hid = word_embeddings.shape
    n_tokens = b * s
    flat_ids = input_ids.reshape(n_tokens).astype(jnp.int32)

    gathered = _sc_gather(word_embeddings, flat_ids, n_tokens, hid)
    gathered = gathered.reshape(b, s, hid)

    bs = 2048
    out = pl.pallas_call(
        _ln_body,
        grid=(b, s // bs),
        in_specs=[
            pl.BlockSpec((1, bs, hid), lambda i, j: (i, j, 0)),
            pl.BlockSpec((bs, hid), lambda i, j: (j, 0)),
            pl.BlockSpec((1, hid), lambda i, j: (0, 0)),
            pl.BlockSpec((1, hid), lambda i, j: (0, 0)),
            pl.BlockSpec((1, hid), lambda i, j: (0, 0)),
        ],
        out_specs=pl.BlockSpec((1, bs, hid), lambda i, j: (i, j, 0)),
        out_shape=jax.ShapeDtypeStruct((b, s, hid), jnp.float32),
    )(
        gathered,
        position_embeddings,
        token_type_embeddings[0:1],
        ln_weight.reshape(1, hid),
        ln_bias.reshape(1, hid),
    )
    return out
